# baseline (speedup 1.0000x reference)
import functools

import jax
import jax.numpy as jnp
from jax import lax
from jax.experimental import pallas as pl
from jax.experimental.pallas import tpu as pltpu
from jax.experimental.pallas import tpu_sc as plsc

_NUM_WORKERS = 32
_L = 200
_BB = 8
_D = 64
_NQ = _L * _BB // 4
_MAXQ = 13
_RING = 3


def _make_gather(v, idx_len):
  mesh = plsc.VectorSubcoreMesh(core_axis_name="c", subcore_axis_name="s")

  @functools.partial(
      pl.kernel,
      mesh=mesh,
      out_type=jax.ShapeDtypeStruct((_L, 8, _BB, 8, 128), jnp.float32),
      scratch_types=[
          pltpu.VMEM((_MAXQ * 512,), jnp.int32),
          pltpu.VMEM((_RING, 256, _D), jnp.float32),
          pltpu.VMEM((2, 8, 4, 8, 128), jnp.float32),
          pltpu.SemaphoreType.DMA((_RING,)),
          pltpu.SemaphoreType.DMA((2,)),
      ],
      compiler_params=pltpu.CompilerParams(
          use_tc_tiling_on_sc=False, needs_layout_passes=False),
  )
  def gather(table_hbm, idx_hbm, out_hbm, idx_v, rows, tbuf, gsem, ssem):
    wid = lax.axis_index("s") * 2 + lax.axis_index("c")
    qs = 12 * wid + jnp.minimum(wid, 16)
    nq = jnp.where(wid < 16, 13, 12)
    npair = 4 * nq
    nchunk = 2 * nq
    iota16 = lax.iota(jnp.int32, 16)
    m15 = jnp.full((16,), 15, jnp.int32)
    m7 = jnp.full((16,), 7, jnp.int32)
    bl8 = [iota16 + 16 * blk for blk in range(8)]

    pltpu.sync_copy(idx_hbm.at[pl.ds(qs * 512, _MAXQ * 512)], idx_v)

    def gather_chunk(c):
      slot = lax.rem(c, _RING)
      return pltpu.make_async_copy(
          table_hbm.at[idx_v.at[pl.ds(c * 256, 256)]],
          rows.at[slot], gsem.at[slot])

    for c in range(_RING - 1):
      gather_chunk(c).start()

    def body(k, carry):
      ql = k // 4
      t = lax.rem(k, 4)
      c = k // 2
      tslot = lax.rem(ql, 2)
      q = qs + ql
      l = q // 2
      b0 = 4 * lax.rem(q, 2)

      @pl.when(lax.rem(k, 2) == 0)
      def _():
        gather_chunk(c).wait()

        @pl.when(c + _RING - 1 < nchunk)
        def _():
          gather_chunk(c + _RING - 1).start()

      @pl.when((t == 0) & (ql >= 2))
      def _():
        pltpu.make_async_copy(
            tbuf.at[tslot], out_hbm.at[l, :, pl.ds(b0, 4)],
            ssem.at[tslot]).wait()

      rslot = lax.rem(c, _RING)
      r0 = 128 * lax.rem(k, 2)
      rblock = rows.at[rslot].at[pl.ds(r0, 128), :]
      tv = jnp.full((16,), t, jnp.int32)

      def load_diag(r):
        m = lax.bitwise_and(iota16 + r, m15)
        return [
            plsc.load_gather(rblock, [bl8[blk], m + 16 * cb])
            for cb in range(4)
            for blk in range(8)
        ]

      def scatter_diag(r, vecs):
        m = lax.bitwise_and(iota16 + r, m15)
        dh0 = lax.shift_right_logical(m, 3)
        dlv = lax.bitwise_and(m, m7)
        for cb in range(4):
          dhv = dh0 + 2 * cb
          for blk in range(8):
            plsc.store_scatter(
                tbuf.at[tslot], [dhv, tv, dlv, bl8[blk]],
                vecs[cb * 8 + blk])

      def rbody(r, vecs):
        nxt = load_diag(r + 1)
        scatter_diag(r, vecs)
        return tuple(nxt)

      last = lax.fori_loop(0, 15, rbody, tuple(load_diag(0)))
      scatter_diag(15, last)

      @pl.when(t == 3)
      def _():
        pltpu.async_copy(
            tbuf.at[tslot], out_hbm.at[l, :, pl.ds(b0, 4)], ssem.at[tslot])
      return carry

    lax.fori_loop(0, npair, body, 0)

    for back in (2, 1):
      ql = nq - back
      q = qs + ql
      pltpu.make_async_copy(
          tbuf.at[lax.rem(ql, 2)],
          out_hbm.at[q // 2, :, pl.ds(4 * lax.rem(q, 2), 4)],
          ssem.at[lax.rem(ql, 2)]).wait()

  return gather


def kernel(x, pe):
  b, l = x.shape
  v = pe.shape[0]
  d = pe.shape[-1]
  total = b * l
  idx_len = total + 512
  xt = jnp.pad(x.T.reshape(total), (0, idx_len - total))
  table = pe.reshape(v, d)
  a = _make_gather(v, idx_len)(table, xt)
  return a.transpose(2, 4, 0, 1, 3).reshape(b, l, 1, d)


# speedup vs baseline: 1.7159x; 1.7159x over previous
import functools

import jax
import jax.numpy as jnp
from jax import lax
from jax.experimental import pallas as pl
from jax.experimental.pallas import tpu as pltpu
from jax.experimental.pallas import tpu_sc as plsc

_NUM_WORKERS = 32
_L = 200
_BB = 8
_D = 64
_NP = _L * _BB
_PPW = _NP // _NUM_WORKERS
_PPC = 5
_NCH = _PPW // _PPC
_P = 129
_NS = 4


def _make_gather(v):
  mesh = plsc.VectorSubcoreMesh(core_axis_name="c", subcore_axis_name="s")

  @functools.partial(
      pl.kernel,
      mesh=mesh,
      out_type=jax.ShapeDtypeStruct((_L, 8, _BB, 8, 128), jnp.float32),
      scratch_types=[
          pltpu.VMEM((_PPW * 128,), jnp.int32),
          pltpu.VMEM((2, _PPC * 128, _D), jnp.float32),
          pltpu.VMEM((_NS, 8, 8, _P), jnp.float32),
          pltpu.SemaphoreType.DMA((2,)),
          pltpu.SemaphoreType.DMA((_NS,)),
      ],
      compiler_params=pltpu.CompilerParams(
          use_tc_tiling_on_sc=False, needs_layout_passes=False),
  )
  def gather(table_hbm, idx_hbm, out_hbm, idx_v, rows, tbuf, gsem, ssem):
    wid = lax.axis_index("s") * 2 + lax.axis_index("c")
    p_base = wid * _PPW
    iota16 = lax.iota(jnp.int32, 16)
    dh0 = lax.shift_right_logical(iota16, 3)
    dlv = lax.bitwise_and(iota16, jnp.full((16,), 7, jnp.int32))
    dhv = [dh0 + 2 * k for k in range(4)]

    pltpu.sync_copy(idx_hbm.at[pl.ds(p_base * 128, _PPW * 128)], idx_v)

    def issue_gather(c):
      par = lax.rem(c, 2)
      pltpu.async_copy(
          table_hbm.at[idx_v.at[pl.ds(c * (_PPC * 128), _PPC * 128)]],
          rows.at[par], gsem.at[par])

    issue_gather(0)

    def body(k, carry):
      c = k // _PPC
      j = lax.rem(k, _PPC)
      par = lax.rem(c, 2)
      tpar = lax.rem(k, _NS)
      p = p_base + k
      l = p // _BB
      bh = lax.rem(p, _BB)

      @pl.when(j == 0)
      def _():
        pltpu.make_async_copy(
            table_hbm.at[idx_v.at[pl.ds(c * (_PPC * 128), _PPC * 128)]],
            rows.at[par], gsem.at[par]).wait()

        @pl.when(c + 1 < _NCH)
        def _():
          issue_gather(c + 1)

      @pl.when(k >= _NS)
      def _():
        pltpu.make_async_copy(
            tbuf.at[tpar, :, :, pl.ds(0, 128)], out_hbm.at[l, :, bh],
            ssem.at[tpar]).wait()

      row0 = j * 128
      for b in range(128):
        bv = jnp.full((16,), b, jnp.int32)
        vecs = [rows[par, row0 + b, pl.ds(16 * k2, 16)] for k2 in range(4)]
        for k2 in range(4):
          plsc.store_scatter(tbuf.at[tpar], [dhv[k2], dlv, bv], vecs[k2])

      pltpu.async_copy(
          tbuf.at[tpar, :, :, pl.ds(0, 128)], out_hbm.at[l, :, bh],
          ssem.at[tpar])
      return carry

    lax.fori_loop(0, _PPW, body, 0)

    for k in range(_PPW - _NS, _PPW):
      p = p_base + k
      pltpu.make_async_copy(
          tbuf.at[lax.rem(k, _NS), :, :, pl.ds(0, 128)],
          out_hbm.at[p // _BB, :, lax.rem(p, _BB)],
          ssem.at[lax.rem(k, _NS)]).wait()

  return gather


def kernel(x, pe):
  b, l = x.shape
  v = pe.shape[0]
  d = pe.shape[-1]
  xt = x.T.reshape(b * l)
  table = pe.reshape(v, d)
  a = _make_gather(v)(table, xt)
  return a.transpose(2, 4, 0, 1, 3).reshape(b, l, 1, d)
